# baseline (speedup 1.0000x reference)
import functools

import jax
import jax.numpy as jnp
from jax import lax
from jax.experimental import pallas as pl
from jax.experimental.pallas import tpu as pltpu
from jax.experimental.pallas import tpu_sc as plsc

B = 16384
D = 16
NNEG = 20
NC, NS, L = 2, 16, 16
NW = NC * NS
BPW = B // NW
E = 128
NCH = BPW // E
GPC = E // L
GIDX = 128

_f32 = jnp.float32
_i32 = jnp.int32


def _dcol(d):
    return jnp.full((L,), d, dtype=_i32)


def _sc_body(cidx, xidx, nidx, ctab, xtab, pos_hbm, negr_hbm,
             idx_c, idx_x, idx_n, crows, xrows, nrows, pos_v, neg_v, sem):
    wid = lax.axis_index("s") * NC + lax.axis_index("c")
    base = wid * BPW
    pltpu.sync_copy(cidx.at[pl.ds(base, BPW)], idx_c)
    pltpu.sync_copy(xidx.at[pl.ds(base, BPW)], idx_x)
    pltpu.sync_copy(nidx.at[pl.ds(base * NNEG, BPW * NNEG)], idx_n)

    for ch in range(NCH):
        cps = [
            pltpu.async_copy(ctab.at[idx_c.at[pl.ds(ch * E, E)]], crows, sem),
            pltpu.async_copy(xtab.at[idx_x.at[pl.ds(ch * E, E)]], xrows, sem),
        ]
        for k in range(E * NNEG // GIDX):
            cps.append(pltpu.async_copy(
                xtab.at[idx_n.at[pl.ds(ch * E * NNEG + k * GIDX, GIDX)]],
                nrows.at[pl.ds(k * GIDX, GIDX)], sem))
        for cp in cps:
            cp.wait()

        def group(g, _):
            lg = g * L + lax.iota(_i32, L)
            off = ch * E + g * L
            cd = [plsc.load_gather(crows, [lg, _dcol(d)]) for d in range(D)]
            pos = plsc.load_gather(xrows, [lg, _dcol(0)]) * cd[0]
            for d in range(1, D):
                pos = pos + plsc.load_gather(xrows, [lg, _dcol(d)]) * cd[d]
            pos_v[pl.ds(off, L)] = pos
            rowb = lg * NNEG
            for j in range(NNEG):
                rj = rowb + j
                acc = plsc.load_gather(nrows, [rj, _dcol(0)]) * cd[0]
                for d in range(1, D):
                    acc = acc + plsc.load_gather(nrows, [rj, _dcol(d)]) * cd[d]
                neg_v[j, pl.ds(off, L)] = acc
            return _

        lax.fori_loop(0, GPC, group, None)

    pltpu.sync_copy(pos_v, pos_hbm.at[pl.ds(base, BPW)])
    pltpu.sync_copy(neg_v, negr_hbm.at[wid])


_sc_dots = functools.partial(
    pl.kernel,
    out_type=(
        jax.ShapeDtypeStruct((B,), _f32),
        jax.ShapeDtypeStruct((NW, NNEG, BPW), _f32),
    ),
    mesh=plsc.VectorSubcoreMesh(core_axis_name="c", subcore_axis_name="s"),
    compiler_params=pltpu.CompilerParams(
        needs_layout_passes=False, use_tc_tiling_on_sc=False),
    scratch_types=[
        pltpu.VMEM((BPW,), _i32),
        pltpu.VMEM((BPW,), _i32),
        pltpu.VMEM((BPW * NNEG,), _i32),
        pltpu.VMEM((E, D), _f32),
        pltpu.VMEM((E, D), _f32),
        pltpu.VMEM((E * NNEG, D), _f32),
        pltpu.VMEM((BPW,), _f32),
        pltpu.VMEM((NNEG, BPW), _f32),
        pltpu.SemaphoreType.DMA,
    ],
)(_sc_body)


def _tc_body(pos_ref, neg_ref, pos_o, neg_o):
    pos_o[...] = jax.nn.log_sigmoid(pos_ref[...])
    x = neg_ref[...]
    ls = jax.nn.log_sigmoid(-x)
    neg_o[...] = ls.reshape(NW, NNEG, BPW).sum(axis=1)


_tc_epilogue = pl.pallas_call(
    _tc_body,
    out_shape=(
        jax.ShapeDtypeStruct((B // 128, 128), _f32),
        jax.ShapeDtypeStruct((NW, BPW), _f32),
    ),
)


def kernel(center_idx, context_idx, negative_idx, center_table, context_table):
    cidx = center_idx.astype(_i32)
    xidx = context_idx.astype(_i32)
    nidx = negative_idx.astype(_i32).reshape(B * NNEG)
    pos_raw, neg_raw = _sc_dots(cidx, xidx, nidx, center_table, context_table)
    pos_ls, neg_s = _tc_epilogue(
        pos_raw.reshape(B // 128, 128),
        neg_raw.reshape(NW * NNEG, BPW),
    )
    return pos_ls.reshape(B), neg_s.reshape(B)
